# Initial kernel scaffold; baseline (speedup 1.0000x reference)
import jax
import jax.numpy as jnp
from jax.experimental import pallas as pl


def kernel(x):
    raise NotImplementedError("write your pallas kernel here")


# speedup vs baseline: 1.3015x; 1.3015x over previous
import jax
import jax.numpy as jnp
from jax.experimental import pallas as pl

NUM_OUT_CHANNELS = 768
CB = 128


def _body(x_ref, o_ref, *, ncopy):
    c = pl.program_id(1)

    @pl.when(c < ncopy)
    def _copy():
        o_ref[...] = x_ref[...]

    @pl.when(c >= ncopy)
    def _zero():
        o_ref[...] = jnp.zeros_like(o_ref)


def kernel(x):
    B, C, H, W = x.shape
    ncopy = C // CB
    ntot = NUM_OUT_CHANNELS // CB

    import functools
    body = functools.partial(_body, ncopy=ncopy)

    return pl.pallas_call(
        body,
        grid=(B, ntot),
        in_specs=[
            pl.BlockSpec(
                (1, CB, H, W),
                lambda b, c: (b, jnp.minimum(c, ncopy - 1), 0, 0),
            )
        ],
        out_specs=pl.BlockSpec((1, CB, H, W), lambda b, c: (b, c, 0, 0)),
        out_shape=jax.ShapeDtypeStruct((B, NUM_OUT_CHANNELS, H, W), x.dtype),
    )(x)
